# baseline (speedup 1.0000x reference)
import numpy as np
import jax
import jax.numpy as jnp
from jax.experimental import pallas as pl
from jax.experimental.pallas import tpu as pltpu

_N = 39
_NP = 1521
_TB = 1024

def _body(r_ref, o_ref):
    o_ref[...] = jnp.broadcast_to(r_ref[0, 0], o_ref.shape)

def kernel(batch_ranking, batch_label):
    del batch_label
    r = jnp.asarray(batch_ranking, jnp.float32).reshape(-1, _N)
    b = r.shape[0]
    chunk = _TB * _NP
    out = pl.pallas_call(
        _body,
        out_shape=jax.ShapeDtypeStruct((b * _NP,), jnp.float32),
        grid=(b // _TB,),
        in_specs=[pl.BlockSpec((_TB, _N), lambda i: (i, 0))],
        out_specs=pl.BlockSpec((chunk,), lambda i: (i,)),
        compiler_params=pltpu.CompilerParams(
            dimension_semantics=("parallel",),
            vmem_limit_bytes=40 << 20,
        ),
    )(r)
    return out.reshape(b, _N, _N)


# speedup vs baseline: 2.7067x; 2.7067x over previous
import numpy as np

import jax
import jax.numpy as jnp
from jax.experimental import pallas as pl
from jax.experimental.pallas import tpu as pltpu

_N = 39
_NP = _N * _N
_NP_PAD = 1536
_TB = 1024
_VMEM_BYTES = 40 << 20


def _pair_diff_const() -> np.ndarray:
    eye = np.eye(_N, dtype=np.float32)
    d = (eye[:, :, None] - eye[:, None, :]).reshape(_N, _NP)
    d_pad = np.zeros((_N, _NP_PAD), dtype=np.float32)
    d_pad[:, :_NP] = d
    return d_pad.astype(np.dtype("bfloat16"))


def _pair_diff_body(r_ref, d_ref, o_ref):
    r16 = r_ref[...].astype(jnp.bfloat16)
    o_ref[...] = jax.lax.dot_general(
        r16, d_ref[...],
        dimension_numbers=(((1,), (0,)), ((), ())),
        preferred_element_type=jnp.float32,
    )


def kernel(batch_ranking, batch_label):
    del batch_label
    r = jnp.asarray(batch_ranking, jnp.float32).reshape(-1, _N)
    b_total = r.shape[0]

    tb = min(_TB, b_total)
    if b_total >= 16:
        half = -(-b_total // 2)
        tb = min(tb, ((half + 7) // 8) * 8)
    grid = (pl.cdiv(b_total, tb),)

    d = jnp.asarray(_pair_diff_const())

    out = pl.pallas_call(
        _pair_diff_body,
        out_shape=jax.ShapeDtypeStruct((b_total, _NP_PAD), jnp.float32),
        grid=grid,
        in_specs=[
            pl.BlockSpec((tb, _N), lambda i: (i, 0)),
            pl.BlockSpec((_N, _NP_PAD), lambda i: (0, 0)),
        ],
        out_specs=pl.BlockSpec((tb, _NP_PAD), lambda i: (i, 0)),
        compiler_params=pltpu.CompilerParams(
            dimension_semantics=("parallel",),
            vmem_limit_bytes=_VMEM_BYTES,
        ),
        cost_estimate=pl.CostEstimate(
            flops=2 * b_total * _N * _NP_PAD,
            transcendentals=0,
            bytes_accessed=b_total * _N * 4 + _N * _NP_PAD * 2
            + b_total * _NP_PAD * 4,
        ),
    )(r, d)

    return out[:, :_NP].reshape(b_total, _N, _N)


# speedup vs baseline: 2.7227x; 1.0059x over previous
import numpy as np

import jax
import jax.numpy as jnp
from jax.experimental import pallas as pl
from jax.experimental.pallas import tpu as pltpu

_N = 39
_NP = _N * _N
_NP_PAD = 1536
_TB = 2048
_VMEM_BYTES = 40 << 20


def _pair_diff_const() -> np.ndarray:
    eye = np.eye(_N, dtype=np.float32)
    d = (eye[:, :, None] - eye[:, None, :]).reshape(_N, _NP)
    d_pad = np.zeros((_N, _NP_PAD), dtype=np.float32)
    d_pad[:, :_NP] = d
    return d_pad.astype(np.dtype("bfloat16"))


def _pair_diff_body(r_ref, d_ref, o_ref):
    r16 = r_ref[...].astype(jnp.bfloat16)
    o_ref[...] = jax.lax.dot_general(
        r16, d_ref[...],
        dimension_numbers=(((1,), (0,)), ((), ())),
        preferred_element_type=jnp.float32,
    )


def kernel(batch_ranking, batch_label):
    del batch_label
    r = jnp.asarray(batch_ranking, jnp.float32).reshape(-1, _N)
    b_total = r.shape[0]

    tb = min(_TB, b_total)
    if b_total >= 16:
        half = -(-b_total // 2)
        tb = min(tb, ((half + 7) // 8) * 8)
    grid = (pl.cdiv(b_total, tb),)

    d = jnp.asarray(_pair_diff_const())

    out = pl.pallas_call(
        _pair_diff_body,
        out_shape=jax.ShapeDtypeStruct((b_total, _NP_PAD), jnp.float32),
        grid=grid,
        in_specs=[
            pl.BlockSpec((tb, _N), lambda i: (i, 0)),
            pl.BlockSpec((_N, _NP_PAD), lambda i: (0, 0)),
        ],
        out_specs=pl.BlockSpec((tb, _NP_PAD), lambda i: (i, 0)),
        compiler_params=pltpu.CompilerParams(
            dimension_semantics=("parallel",),
            vmem_limit_bytes=_VMEM_BYTES,
        ),
        cost_estimate=pl.CostEstimate(
            flops=2 * b_total * _N * _NP_PAD,
            transcendentals=0,
            bytes_accessed=b_total * _N * 4 + _N * _NP_PAD * 2
            + b_total * _NP_PAD * 4,
        ),
    )(r, d)

    return out[:, :_NP].reshape(b_total, _N, _N)


# speedup vs baseline: 2.7228x; 1.0000x over previous
import numpy as np

import jax
import jax.numpy as jnp
from jax.experimental import pallas as pl
from jax.experimental.pallas import tpu as pltpu

_N = 39
_NP = _N * _N
_NP_PAD = 1536
_TB = 4096
_VMEM_BYTES = 60 << 20


def _pair_diff_const() -> np.ndarray:
    eye = np.eye(_N, dtype=np.float32)
    d = (eye[:, :, None] - eye[:, None, :]).reshape(_N, _NP)
    d_pad = np.zeros((_N, _NP_PAD), dtype=np.float32)
    d_pad[:, :_NP] = d
    return d_pad.astype(np.dtype("bfloat16"))


def _pair_diff_body(r_ref, d_ref, o_ref):
    r16 = r_ref[...].astype(jnp.bfloat16)
    o_ref[...] = jax.lax.dot_general(
        r16, d_ref[...],
        dimension_numbers=(((1,), (0,)), ((), ())),
        preferred_element_type=jnp.float32,
    )


def kernel(batch_ranking, batch_label):
    del batch_label
    r = jnp.asarray(batch_ranking, jnp.float32).reshape(-1, _N)
    b_total = r.shape[0]

    tb = min(_TB, b_total)
    if b_total >= 16:
        half = -(-b_total // 2)
        tb = min(tb, ((half + 7) // 8) * 8)
    grid = (pl.cdiv(b_total, tb),)

    d = jnp.asarray(_pair_diff_const())

    out = pl.pallas_call(
        _pair_diff_body,
        out_shape=jax.ShapeDtypeStruct((b_total, _NP_PAD), jnp.float32),
        grid=grid,
        in_specs=[
            pl.BlockSpec((tb, _N), lambda i: (i, 0)),
            pl.BlockSpec((_N, _NP_PAD), lambda i: (0, 0)),
        ],
        out_specs=pl.BlockSpec((tb, _NP_PAD), lambda i: (i, 0)),
        compiler_params=pltpu.CompilerParams(
            dimension_semantics=("parallel",),
            vmem_limit_bytes=_VMEM_BYTES,
        ),
        cost_estimate=pl.CostEstimate(
            flops=2 * b_total * _N * _NP_PAD,
            transcendentals=0,
            bytes_accessed=b_total * _N * 4 + _N * _NP_PAD * 2
            + b_total * _NP_PAD * 4,
        ),
    )(r, d)

    return out[:, :_NP].reshape(b_total, _N, _N)
